# baseline (speedup 1.0000x reference)
import functools

import jax
import jax.numpy as jnp
from jax import lax
from jax.experimental import pallas as pl
from jax.experimental.pallas import tpu as pltpu
from jax.experimental.pallas import tpu_sc as plsc

MASK_ID = 600
KP = 640
WPAD = 64
NEG = -1e20


def _dot(a, b, dims):
    return lax.dot_general(a, b, (dims, ((), ())),
                           preferred_element_type=jnp.float32)


def _score_body(q_ref, k_ref, wq_ref, wk_ref, s_ref, kp_ref):
    @pl.when(pl.program_id(1) == 0)
    def _():
        kp_ref[...] = _dot(k_ref[0], wk_ref[...], ((1,), (1,)))

    qp = _dot(q_ref[0], wq_ref[...], ((1,), (1,)))
    s_ref[...] = _dot(qp, kp_ref[...], ((1,), (1,)))


def _out_body(a_ref, v_ref, wv_ref, wf_ref, bf_ref, o_ref):
    vp = _dot(v_ref[0], wv_ref[...], ((1,), (1,)))
    x = _dot(a_ref[...], vp, ((1,), (0,)))
    o_ref[0] = _dot(x, wf_ref[...], ((1,), (1,))) + bf_ref[...]


def _make_sc_attend(n_rows, w):
    info = plsc.get_sparse_core_info()
    nc, ns = info.num_cores, info.num_subcores
    nw = nc * ns
    rows_per = n_rows // nw

    mesh = plsc.VectorSubcoreMesh(core_axis_name="c", subcore_axis_name="s")

    n_g = rows_per // 16
    n_t = n_g // 2

    scratch = [
        pltpu.VMEM((rows_per, KP), jnp.float32),
        pltpu.VMEM((rows_per * w,), jnp.int32),
        pltpu.VMEM((rows_per * w,), jnp.float32),
        pltpu.VMEM((rows_per, KP), jnp.float32),
        pltpu.VMEM((w * 16,), jnp.float32),
        pltpu.SemaphoreType.DMA,
        pltpu.SemaphoreType.DMA,
        pltpu.SemaphoreType.DMA,
        pltpu.SemaphoreType.DMA,
    ]

    @functools.partial(
        pl.kernel,
        mesh=mesh,
        compiler_params=pltpu.CompilerParams(needs_layout_passes=False),
        out_type=[
            jax.ShapeDtypeStruct((n_rows * w,), jnp.float32),
            jax.ShapeDtypeStruct((n_rows, KP), jnp.float32),
        ],
        scratch_types=scratch,
    )
    def sc_attend(s_hbm, idx_hbm, attn_hbm, a_hbm, s_v, idx_v, attn_v, a_v,
                  e_buf, sem_e, sem_o, a_sem, at_sem):
        wid = lax.axis_index("s") * nc + lax.axis_index("c")
        rbase = wid * rows_per

        lane = lax.iota(jnp.int32, 16)
        zero16 = jnp.zeros((16,), jnp.float32)
        neg16 = jnp.full((16,), NEG, jnp.float32)
        one16 = jnp.ones((16,), jnp.float32)

        def s_copy(g, sem):
            return pltpu.make_async_copy(
                s_hbm.at[pl.ds(rbase + g * 16, 16)],
                s_v.at[pl.ds(g * 16, 16)], sem)

        def a_copy(g):
            return pltpu.make_async_copy(
                a_v.at[pl.ds(g * 16, 16)],
                a_hbm.at[pl.ds(rbase + g * 16, 16)], a_sem)

        def at_copy(g):
            return pltpu.make_async_copy(
                attn_v.at[pl.ds(g * 16 * w, 16 * w)],
                attn_hbm.at[pl.ds((rbase + g * 16) * w, 16 * w)], at_sem)

        s_copy(0, sem_e).start()
        s_copy(1, sem_o).start()
        pltpu.sync_copy(idx_hbm.at[pl.ds(rbase * w, rows_per * w)], idx_v)

        def group(g, sem):
            def zbody(r, c):
                for u in range(KP // 16):
                    a_v[g * 16 + r, pl.ds(u * 16, 16)] = zero16
                return c

            lax.fori_loop(0, 16, zbody, 0)
            s_copy(g, sem).wait()

            row16 = g * 16 + lane
            idx_off = row16 * w

            def pass1(wi, carry):
                mx, sm = carry
                iw = plsc.load_gather(idx_v, [idx_off + wi])
                e = plsc.load_gather(s_v, [row16, iw])
                e = jnp.where(iw == MASK_ID, neg16, e)
                e_buf[pl.ds(wi * 16, 16)] = e
                nmx = jnp.maximum(mx, e)
                sm = sm * jnp.exp(mx - nmx) + jnp.exp(e - nmx)
                return nmx, sm

            mx, sm = lax.fori_loop(0, w, pass1, (neg16, zero16))
            inv = one16 / sm

            def pass2(wi, carry):
                e = e_buf[pl.ds(wi * 16, 16)]
                aw = jnp.exp(e - mx) * inv
                iw = plsc.load_gather(idx_v, [idx_off + wi])
                plsc.store_scatter(attn_v, [idx_off + wi], aw)
                plsc.addupdate_scatter(a_v, [row16, iw], aw)
                return carry

            lax.fori_loop(0, w, pass2, 0)
            a_copy(g).start()
            at_copy(g).start()

        def tbody(t, c):
            g0 = 2 * t

            group(g0, sem_e)

            @pl.when(t + 1 < n_t)
            def _():
                s_copy(g0 + 2, sem_e).start()

            group(g0 + 1, sem_o)

            @pl.when(t + 1 < n_t)
            def _():
                s_copy(g0 + 3, sem_o).start()

            return c

        lax.fori_loop(0, n_t, tbody, 0)

        def drain(g, c):
            a_copy(g).wait()
            at_copy(g).wait()
            return c

        lax.fori_loop(0, n_g, drain, 0)

    return sc_attend


def kernel(value, key, query, mask_ori, Wv, Wk, Wq, Wf, bf):
    n, l, hd = query.shape
    vd = value.shape[2]
    w = mask_ori.shape[2]
    nl = n * l

    lb = 256
    nb = l // lb
    scores = pl.pallas_call(
        _score_body,
        grid=(n, nb),
        in_specs=[
            pl.BlockSpec((1, lb, hd), lambda i, j: (i, j, 0)),
            pl.BlockSpec((1, KP, hd), lambda i, j: (i, 0, 0)),
            pl.BlockSpec((hd, hd), lambda i, j: (0, 0)),
            pl.BlockSpec((hd, hd), lambda i, j: (0, 0)),
        ],
        out_specs=pl.BlockSpec((lb, KP), lambda i, j: (i * nb + j, 0)),
        out_shape=jax.ShapeDtypeStruct((nl, KP), jnp.float32),
        scratch_shapes=[pltpu.VMEM((KP, hd), jnp.float32)],
    )(query, key, Wq, Wk)

    attn_flat, a_dense = _make_sc_attend(nl, w)(
        scores, mask_ori.reshape(nl * w))

    attn = attn_flat.reshape(n, l, w)

    out = pl.pallas_call(
        _out_body,
        grid=(n,),
        in_specs=[
            pl.BlockSpec((l, KP), lambda i: (i, 0)),
            pl.BlockSpec((1, KP, vd), lambda i: (i, 0, 0)),
            pl.BlockSpec((vd, vd), lambda i: (0, 0)),
            pl.BlockSpec((vd, vd), lambda i: (0, 0)),
            pl.BlockSpec((1, vd), lambda i: (0, 0)),
        ],
        out_specs=pl.BlockSpec((1, l, vd), lambda i: (i, 0, 0)),
        out_shape=jax.ShapeDtypeStruct((n, l, vd), jnp.float32),
    )(a_dense, value, Wv, Wf, bf.reshape(1, vd))

    return out, attn


# speedup vs baseline: 1.0537x; 1.0537x over previous
import functools

import jax
import jax.numpy as jnp
from jax import lax
from jax.experimental import pallas as pl
from jax.experimental.pallas import tpu as pltpu
from jax.experimental.pallas import tpu_sc as plsc

MASK_ID = 600
KP = 640
WPAD = 64
NEG = -1e20


def _dot(a, b, dims):
    return lax.dot_general(a, b, (dims, ((), ())),
                           preferred_element_type=jnp.float32)


def _score_body(q_ref, k_ref, wq_ref, wk_ref, s_ref):
    qp = _dot(q_ref[0], wq_ref[...], ((1,), (1,)))
    kp = _dot(k_ref[0], wk_ref[...], ((1,), (1,)))
    s_ref[...] = _dot(qp, kp, ((1,), (1,)))


def _vp_body(v_ref, wv_ref, vp_ref):
    vp_ref[0] = _dot(v_ref[0], wv_ref[...], ((1,), (1,)))


def _out_body(a_ref, vp_ref, wf_ref, bf_ref, o_ref):
    x = _dot(a_ref[...], vp_ref[0], ((1,), (0,)))
    o_ref[0] = _dot(x, wf_ref[...], ((1,), (1,))) + bf_ref[...]


def _make_sc_attend(n_rows, w):
    info = plsc.get_sparse_core_info()
    nc, ns = info.num_cores, info.num_subcores
    nw = nc * ns
    rows_per = n_rows // nw

    mesh = plsc.VectorSubcoreMesh(core_axis_name="c", subcore_axis_name="s")

    n_g = rows_per // 16
    n_t = n_g // 2

    scratch = [
        pltpu.VMEM((rows_per, KP), jnp.float32),
        pltpu.VMEM((rows_per * w,), jnp.int32),
        pltpu.VMEM((rows_per * w,), jnp.float32),
        pltpu.VMEM((rows_per, KP), jnp.float32),
        pltpu.VMEM((w * 16,), jnp.float32),
        pltpu.SemaphoreType.DMA,
        pltpu.SemaphoreType.DMA,
        pltpu.SemaphoreType.DMA,
        pltpu.SemaphoreType.DMA,
    ]

    @functools.partial(
        pl.kernel,
        mesh=mesh,
        compiler_params=pltpu.CompilerParams(needs_layout_passes=False),
        out_type=[
            jax.ShapeDtypeStruct((n_rows * w,), jnp.float32),
            jax.ShapeDtypeStruct((n_rows, KP), jnp.float32),
        ],
        scratch_types=scratch,
    )
    def sc_attend(s_hbm, idx_hbm, attn_hbm, a_hbm, s_v, idx_v, attn_v, a_v,
                  e_buf, sem_e, sem_o, a_sem, at_sem):
        wid = lax.axis_index("s") * nc + lax.axis_index("c")
        rbase = wid * rows_per

        lane = lax.iota(jnp.int32, 16)
        zero16 = jnp.zeros((16,), jnp.float32)
        neg16 = jnp.full((16,), NEG, jnp.float32)
        one16 = jnp.ones((16,), jnp.float32)

        def s_copy(g, sem):
            return pltpu.make_async_copy(
                s_hbm.at[pl.ds(rbase + g * 16, 16)],
                s_v.at[pl.ds(g * 16, 16)], sem)

        def a_copy(g):
            return pltpu.make_async_copy(
                a_v.at[pl.ds(g * 16, 16)],
                a_hbm.at[pl.ds(rbase + g * 16, 16)], a_sem)

        def at_copy(g):
            return pltpu.make_async_copy(
                attn_v.at[pl.ds(g * 16 * w, 16 * w)],
                attn_hbm.at[pl.ds((rbase + g * 16) * w, 16 * w)], at_sem)

        s_copy(0, sem_e).start()
        s_copy(1, sem_o).start()
        pltpu.sync_copy(idx_hbm.at[pl.ds(rbase * w, rows_per * w)], idx_v)

        def group(g, sem):
            def zbody(r, c):
                for u in range(KP // 16):
                    a_v[g * 16 + r, pl.ds(u * 16, 16)] = zero16
                return c

            lax.fori_loop(0, 16, zbody, 0)
            s_copy(g, sem).wait()

            row16 = g * 16 + lane
            idx_off = row16 * w

            def pass1(wi, carry):
                mx, sm = carry
                iw = plsc.load_gather(idx_v, [idx_off + wi])
                e = plsc.load_gather(s_v, [row16, iw])
                e = jnp.where(iw == MASK_ID, neg16, e)
                e_buf[pl.ds(wi * 16, 16)] = e
                nmx = jnp.maximum(mx, e)
                sm = sm * jnp.exp(mx - nmx) + jnp.exp(e - nmx)
                return nmx, sm

            mx, sm = lax.fori_loop(0, w, pass1, (neg16, zero16))
            inv = one16 / sm

            def pass2(wi, carry):
                e = e_buf[pl.ds(wi * 16, 16)]
                aw = jnp.exp(e - mx) * inv
                iw = plsc.load_gather(idx_v, [idx_off + wi])
                plsc.store_scatter(attn_v, [idx_off + wi], aw)
                plsc.addupdate_scatter(a_v, [row16, iw], aw)
                return carry

            lax.fori_loop(0, w, pass2, 0)
            a_copy(g).start()
            at_copy(g).start()

        def tbody(t, c):
            g0 = 2 * t

            group(g0, sem_e)

            @pl.when(t + 1 < n_t)
            def _():
                s_copy(g0 + 2, sem_e).start()

            group(g0 + 1, sem_o)

            @pl.when(t + 1 < n_t)
            def _():
                s_copy(g0 + 3, sem_o).start()

            return c

        lax.fori_loop(0, n_t, tbody, 0)

        def drain(g, c):
            a_copy(g).wait()
            at_copy(g).wait()
            return c

        lax.fori_loop(0, n_g, drain, 0)

    return sc_attend


def kernel(value, key, query, mask_ori, Wv, Wk, Wq, Wf, bf):
    n, l, hd = query.shape
    vd = value.shape[2]
    w = mask_ori.shape[2]
    nl = n * l

    scores = pl.pallas_call(
        _score_body,
        grid=(n,),
        in_specs=[
            pl.BlockSpec((1, l, hd), lambda i: (i, 0, 0)),
            pl.BlockSpec((1, KP, hd), lambda i: (i, 0, 0)),
            pl.BlockSpec((hd, hd), lambda i: (0, 0)),
            pl.BlockSpec((hd, hd), lambda i: (0, 0)),
        ],
        out_specs=pl.BlockSpec((l, KP), lambda i: (i, 0)),
        out_shape=jax.ShapeDtypeStruct((nl, KP), jnp.float32),
    )(query, key, Wq, Wk)

    attn_flat, a_dense = _make_sc_attend(nl, w)(
        scores, mask_ori.reshape(nl * w))

    attn = attn_flat.reshape(n, l, w)

    vp = pl.pallas_call(
        _vp_body,
        grid=(n,),
        in_specs=[
            pl.BlockSpec((1, KP, vd), lambda i: (i, 0, 0)),
            pl.BlockSpec((vd, vd), lambda i: (0, 0)),
        ],
        out_specs=pl.BlockSpec((1, KP, vd), lambda i: (i, 0, 0)),
        out_shape=jax.ShapeDtypeStruct((n, KP, vd), jnp.float32),
    )(value, Wv)

    out = pl.pallas_call(
        _out_body,
        grid=(n,),
        in_specs=[
            pl.BlockSpec((l, KP), lambda i: (i, 0)),
            pl.BlockSpec((1, KP, vd), lambda i: (i, 0, 0)),
            pl.BlockSpec((vd, vd), lambda i: (0, 0)),
            pl.BlockSpec((1, vd), lambda i: (0, 0)),
        ],
        out_specs=pl.BlockSpec((1, l, vd), lambda i: (i, 0, 0)),
        out_shape=jax.ShapeDtypeStruct((n, l, vd), jnp.float32),
    )(a_dense, vp, Wf, bf.reshape(1, vd))

    return out, attn
